# baseline (speedup 1.0000x reference)
import functools

import jax
import jax.numpy as jnp
from jax.experimental import pallas as pl
from jax.experimental.pallas import tpu as pltpu


def _conv3x3_kernel(x_ref, w_ref, o_ref, buf_ref, *, B, W, L, Mg):
    C = x_ref.shape[1]
    bf16 = jnp.bfloat16
    P = L + Mg
    S = Mg + B * P
    NL = (B - 1) * P + L

    for r in range(2):
        sh = 0 if r == 0 else -W
        rows = slice(r * C, (r + 1) * C)
        buf_ref[rows, :Mg + sh] = jnp.zeros((C, Mg + sh), bf16)
        for b in range(B):
            lo = Mg + b * P + sh
            if b:
                buf_ref[rows, lo - Mg:lo] = jnp.zeros((C, Mg), bf16)
            buf_ref[rows, lo:lo + L] = x_ref[b].astype(bf16)
        hi = Mg + (B - 1) * P + sh + L
        buf_ref[rows, hi:] = jnp.zeros((C, S - hi), bf16)

    col = jax.lax.broadcasted_iota(jnp.int32, (1, NL), 1) % W
    not_first = (col != 0).astype(jnp.float32)
    not_last = (col != W - 1).astype(jnp.float32)

    def mm(nblk, o, wi):
        sl = buf_ref[:nblk * C, Mg + o:Mg + o + NL] if nblk == 2 else \
            buf_ref[C:2 * C, Mg + o:Mg + o + NL]
        wm = w_ref[wi] if nblk == 2 else w_ref[wi, :, :C]
        return jnp.dot(wm, sl, preferred_element_type=jnp.float32)

    left = mm(2, -W - 1, 0) + mm(1, -1, 3)
    mid = mm(2, -W, 1) + mm(1, 0, 4)
    right = mm(2, -W + 1, 2) + mm(1, 1, 5)
    res = (mid + left * not_first + right * not_last).astype(bf16)
    for b in range(B):
        o_ref[b] = res[:, b * P:b * P + L]


def kernel(x, w):
    N, Cin, H, W = x.shape
    Cout, _, K, _ = w.shape
    assert K == 3
    L = H * W
    Mg = 128
    B = 2
    assert N % B == 0

    x_flat = x.reshape(N, Cin, L)

    wt = jnp.transpose(w, (2, 3, 0, 1)).astype(jnp.bfloat16)
    pair = jnp.concatenate([wt[0], wt[1]], axis=2)
    single = jnp.concatenate([wt[2], jnp.zeros_like(wt[2])], axis=2)
    w6 = jnp.stack([pair[0], pair[1], pair[2],
                    single[0], single[1], single[2]])

    out = pl.pallas_call(
        functools.partial(_conv3x3_kernel, B=B, W=W, L=L, Mg=Mg),
        out_shape=jax.ShapeDtypeStruct((N, Cout, L), jnp.bfloat16),
        grid=(N // B,),
        in_specs=[
            pl.BlockSpec((B, Cin, L), lambda n: (n, 0, 0)),
            pl.BlockSpec((6, Cout, 2 * Cin), lambda n: (0, 0, 0)),
        ],
        out_specs=pl.BlockSpec((B, Cout, L), lambda n: (n, 0, 0)),
        scratch_shapes=[pltpu.VMEM((2 * Cin, Mg + B * (L + Mg)),
                                   jnp.bfloat16)],
        compiler_params=pltpu.CompilerParams(
            dimension_semantics=("parallel",)),
    )(x_flat, w6)
    return out.reshape(N, Cout, H, W).astype(jnp.float32)
